# Initial kernel scaffold; baseline (speedup 1.0000x reference)
import jax
import jax.numpy as jnp
from jax.experimental import pallas as pl


def kernel(X, edge_index, batch, W1, b1, gamma, beta, W2, b2, Wc1, bc1, Wc2, bc2, Wc3, bc3):
    raise NotImplementedError("write your pallas kernel here")


# speedup vs baseline: 2.6136x; 2.6136x over previous
import functools

import jax
import jax.numpy as jnp
from jax import lax
from jax.experimental import pallas as pl
from jax.experimental.pallas import tpu as pltpu
from jax.experimental.pallas import tpu_sc as plsc

NC = 2
NS = 16
NW = NC * NS
CK = 2
G = 128


def _edge_agg_call(h, src2d, dst2d, n_pad):
    D = h.shape[1]
    rows_per_w = src2d.shape[0] // NW
    rpt = n_pad // NS
    mesh = plsc.VectorSubcoreMesh(core_axis_name="c", subcore_axis_name="s")

    @functools.partial(
        pl.kernel,
        out_type=jax.ShapeDtypeStruct((NC, n_pad, D), jnp.float32),
        mesh=mesh,
        scratch_types=[
            pltpu.VMEM((CK, 128), jnp.int32),
            pltpu.VMEM((CK, 128), jnp.int32),
            pltpu.VMEM((CK, 128, D), jnp.float32),
            pltpu.VMEM((8, D), jnp.float32),
            pltpu.VMEM_SHARED((n_pad, D), jnp.float32),
            pltpu.SemaphoreType.DMA,
        ],
    )
    def k(h_hbm, src_hbm, dst_hbm, out_hbm, src_v, dst_v, rows_v, zero_v,
          acc_sh, sem):
        c = lax.axis_index("c")
        s = lax.axis_index("s")
        w = c * NS + s
        zv = jnp.zeros((16,), jnp.float32)
        for r in range(8):
            for q in range(D // 16):
                zero_v[r, pl.ds(q * 16, 16)] = zv

        def zrow(i, carry):
            pltpu.sync_copy(zero_v, acc_sh.at[pl.ds(s * rpt + i * 8, 8)])
            return carry
        lax.fori_loop(0, rpt // 8, zrow, 0)
        plsc.subcore_barrier()

        def chunk(g, carry):
            r0 = w * rows_per_w + g * CK
            pltpu.sync_copy(src_hbm.at[pl.ds(r0, CK)], src_v)
            pltpu.sync_copy(dst_hbm.at[pl.ds(r0, CK)], dst_v)
            descs = [
                pltpu.async_copy(h_hbm.at[src_v.at[j]], rows_v.at[j], sem)
                for j in range(CK)
            ]
            for d in descs:
                d.wait()
            for j in range(CK):
                pltpu.sync_copy(rows_v.at[j], acc_sh.at[dst_v.at[j]], add=True)
            return carry
        lax.fori_loop(0, rows_per_w // CK, chunk, 0)
        plsc.subcore_barrier()
        pltpu.sync_copy(acc_sh.at[pl.ds(s * rpt, rpt)],
                        out_hbm.at[c, pl.ds(s * rpt, rpt)])

    return k(h, src2d, dst2d)


def _layer_pre(h, agg, W1l, b1l, tn):
    N, D = h.shape
    nt = N // tn

    def body(h_ref, agg_ref, w_ref, b_ref, z_ref, st_ref):
        i = pl.program_id(0)
        x = h_ref[...] + agg_ref[0] + agg_ref[1]
        z = jnp.dot(x, w_ref[...], preferred_element_type=jnp.float32, precision=lax.Precision.HIGHEST)
        z = z + b_ref[...]
        z_ref[...] = z
        s1 = jnp.sum(z, axis=0, keepdims=True)
        s2 = jnp.sum(z * z, axis=0, keepdims=True)
        st = jnp.concatenate([s1, s2], axis=0)

        @pl.when(i == 0)
        def _():
            st_ref[...] = st

        @pl.when(i > 0)
        def _():
            st_ref[...] += st

    return pl.pallas_call(
        body,
        grid=(nt,),
        in_specs=[
            pl.BlockSpec((tn, D), lambda i: (i, 0)),
            pl.BlockSpec((NC, tn, D), lambda i: (0, i, 0)),
            pl.BlockSpec((D, D), lambda i: (0, 0)),
            pl.BlockSpec((1, D), lambda i: (0, 0)),
        ],
        out_specs=[
            pl.BlockSpec((tn, D), lambda i: (i, 0)),
            pl.BlockSpec((2, D), lambda i: (0, 0)),
        ],
        out_shape=[
            jax.ShapeDtypeStruct((N, D), jnp.float32),
            jax.ShapeDtypeStruct((2, D), jnp.float32),
        ],
    )(h, agg, W1l, b1l)


def _layer_post(z, stats, gammal, betal, W2l, b2l, batch2d, tn):
    N, D = z.shape
    nt = N // tn

    def body(z_ref, st_ref, g_ref, be_ref, w_ref, b_ref, bt_ref, h_ref, p_ref):
        i = pl.program_id(0)
        n = jnp.float32(N)
        mean = st_ref[0:1] / n
        var = st_ref[1:2] / n - mean * mean
        rstd = lax.rsqrt(var + 1e-5)
        zn = (z_ref[...] - mean) * (rstd * g_ref[...]) + be_ref[...]
        a = jnp.maximum(zn, 0.0)
        h2 = jnp.dot(a, w_ref[...], preferred_element_type=jnp.float32, precision=lax.Precision.HIGHEST)
        h2 = jnp.maximum(h2 + b_ref[...], 0.0)
        h_ref[...] = h2
        gidx = bt_ref[...]
        oh = (gidx == lax.broadcasted_iota(jnp.int32, (tn, G), 1))
        oh = oh.astype(jnp.float32)
        pool = lax.dot_general(oh, h2, (((0,), (0,)), ((), ())),
                               preferred_element_type=jnp.float32, precision=lax.Precision.HIGHEST)

        @pl.when(i == 0)
        def _():
            p_ref[...] = pool

        @pl.when(i > 0)
        def _():
            p_ref[...] += pool

    return pl.pallas_call(
        body,
        grid=(nt,),
        in_specs=[
            pl.BlockSpec((tn, D), lambda i: (i, 0)),
            pl.BlockSpec((2, D), lambda i: (0, 0)),
            pl.BlockSpec((1, D), lambda i: (0, 0)),
            pl.BlockSpec((1, D), lambda i: (0, 0)),
            pl.BlockSpec((D, D), lambda i: (0, 0)),
            pl.BlockSpec((1, D), lambda i: (0, 0)),
            pl.BlockSpec((tn, 1), lambda i: (i, 0)),
        ],
        out_specs=[
            pl.BlockSpec((tn, D), lambda i: (i, 0)),
            pl.BlockSpec((G, D), lambda i: (0, 0)),
        ],
        out_shape=[
            jax.ShapeDtypeStruct((N, D), jnp.float32),
            jax.ShapeDtypeStruct((G, D), jnp.float32),
        ],
    )(z, stats, gammal, betal, W2l, b2l, batch2d)


def _head(p1, p2, p3, wa, wb, wc, bc1, Wc2, bc2, Wc3p, bc3p):
    D = p1.shape[1]

    def body(p1r, p2r, p3r, war, wbr, wcr, b1r, w2r, b2r, w3r, b3r, out):
        o = (jnp.dot(p1r[...], war[...], preferred_element_type=jnp.float32, precision=lax.Precision.HIGHEST)
             + jnp.dot(p2r[...], wbr[...], preferred_element_type=jnp.float32, precision=lax.Precision.HIGHEST)
             + jnp.dot(p3r[...], wcr[...], preferred_element_type=jnp.float32, precision=lax.Precision.HIGHEST))
        o = jnp.maximum(o + b1r[...], 0.0)
        o = jnp.dot(o, w2r[...], preferred_element_type=jnp.float32, precision=lax.Precision.HIGHEST)
        o = jnp.maximum(o + b2r[...], 0.0)
        out[...] = jnp.dot(o, w3r[...],
                           preferred_element_type=jnp.float32, precision=lax.Precision.HIGHEST) + b3r[...]

    return pl.pallas_call(
        body,
        out_shape=jax.ShapeDtypeStruct((G, D), jnp.float32),
    )(p1, p2, p3, wa, wb, wc, bc1, Wc2, bc2, Wc3p, bc3p)


def kernel(X, edge_index, batch, W1, b1, gamma, beta, W2, b2,
           Wc1, bc1, Wc2, bc2, Wc3, bc3):
    N, D = X.shape
    E = edge_index.shape[1]
    L = W1.shape[0]

    rows = -(-E // 128)
    rows_pad = -(-rows // (NW * CK)) * (NW * CK)
    e_pad = rows_pad * 128
    n_pad = -(-(N + 1) // 256) * 256
    src = jnp.concatenate(
        [edge_index[0], jnp.zeros((e_pad - E,), jnp.int32)]).reshape(-1, 128)
    dst = jnp.concatenate(
        [edge_index[1], jnp.full((e_pad - E,), N, jnp.int32)]).reshape(-1, 128)
    batch2d = batch.reshape(N, 1)

    tn = 1000
    h = X
    pooled = []
    for l in range(L):
        agg = _edge_agg_call(h, src, dst, n_pad)
        z, stats = _layer_pre(h, agg, W1[l], b1[l].reshape(1, D), tn)
        h, pool = _layer_post(z, stats, gamma[l].reshape(1, D),
                              beta[l].reshape(1, D), W2[l],
                              b2[l].reshape(1, D), batch2d, tn)
        pooled.append(pool)

    wa, wb, wc = Wc1[0:D], Wc1[D:2 * D], Wc1[2 * D:3 * D]
    Wc3p = jnp.pad(Wc3, ((0, 0), (0, D - Wc3.shape[1])))
    bc3p = jnp.pad(bc3, (0, D - bc3.shape[0])).reshape(1, D)
    out = _head(pooled[0], pooled[1], pooled[2], wa, wb, wc,
                bc1.reshape(1, D), Wc2, bc2.reshape(1, D), Wc3p, bc3p)
    return out[:, :1]
